# baseline (speedup 1.0000x reference)
import functools
import math

import jax
import jax.numpy as jnp
from jax import lax
from jax.experimental import pallas as pl
from jax.experimental.pallas import tpu as pltpu
from jax.experimental.pallas import tpu_sc as plsc

T = 20
NW = 32
CH = 128


def _sc_gather_all(us, its, beh,
                   emb_user_id, emb_user_city, emb_user_age,
                   emb_item, emb_cate):
  B = us.shape[0]
  nq = (B // NW) // CH
  mesh = plsc.VectorSubcoreMesh(core_axis_name="c", subcore_axis_name="s")

  out_type = [
      jax.ShapeDtypeStruct((B, 32), jnp.float32),
      jax.ShapeDtypeStruct((B, 32), jnp.float32),
      jax.ShapeDtypeStruct((B, 32), jnp.float32),
      jax.ShapeDtypeStruct((B, 64), jnp.float32),
      jax.ShapeDtypeStruct((B, 64), jnp.float32),
      jax.ShapeDtypeStruct((T * B, 64), jnp.float32),
      jax.ShapeDtypeStruct((T * B, 64), jnp.float32),
      jax.ShapeDtypeStruct((T * B,), jnp.int32),
  ]
  scratch_types = [
      pltpu.VMEM((CH, 3), jnp.int32),
      pltpu.VMEM((CH, 3 * T), jnp.int32),
      pltpu.VMEM((CH,), jnp.int32),
      pltpu.VMEM((CH, 32), jnp.float32),
      pltpu.VMEM((CH, 64), jnp.float32),
      pltpu.SemaphoreType.DMA,
  ]

  @functools.partial(pl.kernel, out_type=out_type, mesh=mesh,
                     scratch_types=scratch_types,
                     compiler_params=pltpu.CompilerParams(
                         use_tc_tiling_on_sc=False,
                         needs_layout_passes=False))
  def k(ush, itsh, behh,
        t_u0, t_u1, t_u2, t_it, t_ct,
        ue0o, ue1o, ue2o, qio, qco, bio, bco, mko,
        s3_v, sb_v, idx_v, rows32, rows64, sem):
    wid = lax.axis_index("s") * 2 + lax.axis_index("c")
    qbase = wid * (B // NW)

    def extract_col(src_v, col):
      for g in range(CH // 16):
        rows = lax.iota(jnp.int32, 16) + g * 16
        cols = jnp.broadcast_to(jnp.int32(0) + col, (16,))
        idx_v[pl.ds(g * 16, 16)] = plsc.load_gather(src_v, [rows, cols])

    def gather_to(table, rows, outh, off):
      pltpu.async_copy(table.at[idx_v], rows, sem).wait()
      pltpu.sync_copy(rows, outh.at[pl.ds(off, CH)])

    for c in range(nq):
      base = qbase + c * CH
      pltpu.sync_copy(ush.at[pl.ds(base, CH)], s3_v)
      extract_col(s3_v, 0)
      gather_to(t_u0, rows32, ue0o, base)
      extract_col(s3_v, 1)
      gather_to(t_u1, rows32, ue1o, base)
      extract_col(s3_v, 2)
      gather_to(t_u2, rows32, ue2o, base)
      pltpu.sync_copy(itsh.at[pl.ds(base, CH)], s3_v)
      extract_col(s3_v, 0)
      gather_to(t_it, rows64, qio, base)
      extract_col(s3_v, 1)
      gather_to(t_ct, rows64, qco, base)
      pltpu.sync_copy(behh.at[pl.ds(base, CH)], sb_v)

      def body(t, carry):
        off = t * B + base
        extract_col(sb_v, 3 * t + 1)
        gather_to(t_it, rows64, bio, off)
        extract_col(sb_v, 3 * t + 2)
        gather_to(t_ct, rows64, bco, off)
        extract_col(sb_v, 3 * t)
        pltpu.sync_copy(idx_v, mko.at[pl.ds(off, CH)])
        return carry
      lax.fori_loop(0, T, body, 0)

  return k(us, its, beh,
           emb_user_id, emb_user_city, emb_user_age, emb_item, emb_cate)


def _prelu(x, a):
  return jnp.where(x >= 0.0, x, a * x)


def _dot(x, w):
  return jnp.dot(x, w, preferred_element_type=jnp.float32)


def _tc_body(Bb,
             ud_r, isf_r, ue0_r, ue1_r, ue2_r, qi_r, qc_r,
             bi_r, bc_r, mk_r,
             WAi_r, WAc_r, WBi_r, WBc_r, WDi_r, WDc_r,
             ab0_r, aa0_r, aW1_r, ab1_r, aa1_r, aWf_r, abf_r,
             Fud_r, Fisf_r, Fue0_r, Fue1_r, Fue2_r, Fqi_r, Fqc_r,
             Fai_r, Fac_r, fb0_r, fa0_r, fW1_r, fb1_r, fa1_r,
             oW_r, ob_r, out_r):
  qi = qi_r[...]
  qc = qc_r[...]
  bi3 = bi_r[...]
  bc3 = bc_r[...]

  bir = bi3.reshape(T * Bb, 64)
  bcr = bc3.reshape(T * Bb, 64)
  pir = (bi3 * qi[None, :, :]).reshape(T * Bb, 64)
  pcr = (bc3 * qc[None, :, :]).reshape(T * Bb, 64)

  hq = _dot(qi, WAi_r[...]) + _dot(qc, WAc_r[...])
  h0 = (jnp.broadcast_to(hq[None], (T, Bb, 80)).reshape(T * Bb, 80)
        + _dot(bir, WBi_r[...]) + _dot(bcr, WBc_r[...])
        + _dot(pir, WDi_r[...]) + _dot(pcr, WDc_r[...]) + ab0_r[...])
  h0 = _prelu(h0, aa0_r[...])
  h1 = _prelu(_dot(h0, aW1_r[...]) + ab1_r[...], aa1_r[...])
  s = _dot(h1, aWf_r[...]) + abf_r[...]
  s3 = s.reshape(T, Bb, 1)
  s3 = jnp.where(mk_r[...] == 0, jnp.float32(-4294967295.0), s3)
  m = jnp.max(s3, axis=0, keepdims=True)
  e = jnp.exp(s3 - m)
  w3 = e / jnp.sum(e, axis=0, keepdims=True)
  atti = jnp.sum(w3 * bi3, axis=0)
  attc = jnp.sum(w3 * bc3, axis=0)

  h2 = (_dot(ud_r[...], Fud_r[...]) + _dot(isf_r[...], Fisf_r[...])
        + _dot(ue0_r[...], Fue0_r[...]) + _dot(ue1_r[...], Fue1_r[...])
        + _dot(ue2_r[...], Fue2_r[...])
        + _dot(qi, Fqi_r[...]) + _dot(qc, Fqc_r[...])
        + _dot(atti, Fai_r[...]) + _dot(attc, Fac_r[...]) + fb0_r[...])
  h2 = _prelu(h2, fa0_r[...])
  h3 = _prelu(_dot(h2, fW1_r[...]) + fb1_r[...], fa1_r[...])
  out_r[...] = jax.nn.sigmoid(_dot(h3, oW_r[...]) + ob_r[...])


def _tc_dense(ud, isf, ue0, ue1, ue2, qi, qc, bi3, bc3, mk3, weights,
              interpret=False):
  B = ud.shape[0]
  Bb = 512 if B % 512 == 0 else B
  grid = (B // Bb,)

  def rows(n):
    return pl.BlockSpec((Bb, n), lambda i: (i, 0))

  def full(a):
    return pl.BlockSpec(a.shape, lambda i: (0,) * a.ndim)

  in_specs = [
      rows(5), rows(3), rows(32), rows(32), rows(32), rows(64), rows(64),
      pl.BlockSpec((T, Bb, 64), lambda i: (0, i, 0)),
      pl.BlockSpec((T, Bb, 64), lambda i: (0, i, 0)),
      pl.BlockSpec((T, Bb, 1), lambda i: (0, i, 0)),
  ] + [full(w) for w in weights]

  return pl.pallas_call(
      functools.partial(_tc_body, Bb),
      grid=grid,
      in_specs=in_specs,
      out_specs=pl.BlockSpec((Bb, 1), lambda i: (i, 0)),
      out_shape=jax.ShapeDtypeStruct((B, 1), jnp.float32),
      interpret=interpret,
  )(ud, isf, ue0, ue1, ue2, qi, qc, bi3, bc3, mk3, *weights)


def _prep_weights(att_W0, att_b0, att_a0, att_W1, att_b1, att_a1,
                  att_Wf, att_bf, bn_gamma, bn_beta,
                  ffn_W0, ffn_b0, ffn_a0, ffn_W1, ffn_b1, ffn_a1,
                  out_W, out_b):
  A, Bm, C, D = (att_W0[0:128], att_W0[128:256],
                 att_W0[256:384], att_W0[384:512])
  AC = A + C
  BC = Bm - C
  g = bn_gamma / math.sqrt(1.0 + 1e-3)
  F = ffn_W0 * g[:, None]
  fb0 = ffn_b0 + bn_beta @ ffn_W0
  r = lambda v: v.reshape(1, -1)
  return [
      AC[0:64], AC[64:128], BC[0:64], BC[64:128], D[0:64], D[64:128],
      r(att_b0), r(att_a0), att_W1, r(att_b1), r(att_a1), att_Wf, r(att_bf),
      F[0:5], F[101:104], F[5:37], F[37:69], F[69:101],
      F[104:168], F[168:232], F[232:296], F[296:360],
      r(fb0), r(ffn_a0), ffn_W1, r(ffn_b1), r(ffn_a1), out_W, r(out_b),
  ]


def kernel(user_dense_input, user_sparse_input, item_dense_input,
           item_sparse_input, behavior_input, emb_user_id, emb_user_city,
           emb_user_age, emb_item, emb_cate, att_W0, att_b0, att_a0,
           att_W1, att_b1, att_a1, att_Wf, att_bf, bn_gamma, bn_beta,
           ffn_W0, ffn_b0, ffn_a0, ffn_W1, ffn_b1, ffn_a1, out_W, out_b):
  B = user_dense_input.shape[0]
  us = user_sparse_input.astype(jnp.int32)
  its = item_sparse_input.astype(jnp.int32)
  beh = behavior_input.astype(jnp.int32)

  ue0, ue1, ue2, qi, qc, bi, bc, mk = _sc_gather_all(
      us, its, beh,
      emb_user_id, emb_user_city, emb_user_age, emb_item, emb_cate)
  mk3 = mk.reshape(T, B, 1)

  weights = _prep_weights(att_W0, att_b0, att_a0, att_W1, att_b1, att_a1,
                          att_Wf, att_bf, bn_gamma, bn_beta,
                          ffn_W0, ffn_b0, ffn_a0, ffn_W1, ffn_b1, ffn_a1,
                          out_W, out_b)

  return _tc_dense(user_dense_input, its.astype(jnp.float32),
                   ue0, ue1, ue2, qi, qc,
                   bi.reshape(T, B, 64), bc.reshape(T, B, 64), mk3, weights)


# speedup vs baseline: 1.0824x; 1.0824x over previous
import functools
import math

import jax
import jax.numpy as jnp
from jax import lax
from jax.experimental import pallas as pl
from jax.experimental.pallas import tpu as pltpu
from jax.experimental.pallas import tpu_sc as plsc

T = 20
NW = 32
CH = 128


_SC_PARAMS = dict(
    compiler_params=pltpu.CompilerParams(
        use_tc_tiling_on_sc=False, needs_layout_passes=False))
NB = 3


def _extract_col(src_v, col, dst_v):
  for g in range(CH // 16):
    rows = lax.iota(jnp.int32, 16) + g * 16
    cols = jnp.broadcast_to(jnp.int32(0) + col, (16,))
    dst_v[pl.ds(g * 16, 16)] = plsc.load_gather(src_v, [rows, cols])


def _sc_gather_a(us, its, beh,
                 emb_user_id, emb_user_city, emb_user_age, emb_cate):
  B = us.shape[0]
  nq = (B // NW) // CH
  mesh = plsc.VectorSubcoreMesh(core_axis_name="c", subcore_axis_name="s")

  out_type = [
      jax.ShapeDtypeStruct((B, 32), jnp.float32),
      jax.ShapeDtypeStruct((B, 32), jnp.float32),
      jax.ShapeDtypeStruct((B, 32), jnp.float32),
      jax.ShapeDtypeStruct((B, 64), jnp.float32),
      jax.ShapeDtypeStruct((T * B, 64), jnp.float32),
      jax.ShapeDtypeStruct((T * B,), jnp.int32),
  ]
  scratch_types = (
      [pltpu.VMEM((CH, 3), jnp.int32),
       pltpu.VMEM((CH, 3 * T), jnp.int32)]
      + [pltpu.VMEM((CH,), jnp.int32) for _ in range(NB)]
      + [pltpu.VMEM((CH, 32), jnp.float32) for _ in range(NB)]
      + [pltpu.VMEM((CH, 64), jnp.float32) for _ in range(NB)]
      + [pltpu.VMEM((CH,), jnp.int32) for _ in range(2)]
      + [pltpu.SemaphoreType.DMA for _ in range(2 * NB + 2)]
  )

  @functools.partial(pl.kernel, out_type=out_type, mesh=mesh,
                     scratch_types=scratch_types, **_SC_PARAMS)
  def k(ush, itsh, behh, t_u0, t_u1, t_u2, t_ct,
        ue0o, ue1o, ue2o, qco, bco, mko,
        s3_v, sb_v, i0, i1, i2, r320, r321, r322, r640, r641, r642,
        m0, m1, g0, g1, g2, w0, w1, w2, ms0, ms1):
    idx = [i0, i1, i2]
    r32 = [r320, r321, r322]
    r64 = [r640, r641, r642]
    mkb = [m0, m1]
    gs = [g0, g1, g2]
    ws = [w0, w1, w2]
    mss = [ms0, ms1]
    wid = lax.axis_index("s") * 2 + lax.axis_index("c")
    qbase = wid * (B // NW)

    def chunk(c, carry):
      base = qbase + c * CH
      pltpu.sync_copy(ush.at[pl.ds(base, CH)], s3_v)
      tabs = [(t_u0, ue0o), (t_u1, ue1o), (t_u2, ue2o)]
      gd = []
      for f, (tab, _) in enumerate(tabs):
        _extract_col(s3_v, f, idx[f])
        gd.append(pltpu.async_copy(tab.at[idx[f]], r32[f], gs[f]))
      wd = []
      for f, (_, outh) in enumerate(tabs):
        gd[f].wait()
        wd.append(pltpu.async_copy(r32[f], outh.at[pl.ds(base, CH)], ws[f]))
      pltpu.sync_copy(itsh.at[pl.ds(base, CH)], s3_v)
      _extract_col(s3_v, 1, idx[0])
      gq = pltpu.async_copy(t_ct.at[idx[0]], r64[0], gs[0])
      pltpu.sync_copy(behh.at[pl.ds(base, CH)], sb_v)
      for w in wd:
        w.wait()
      gq.wait()
      wq = pltpu.async_copy(r64[0], qco.at[pl.ds(base, CH)], ws[0])
      gb = [None] * NB
      wb = [None] * NB
      wm = [None, None]
      for t in range(T):
        s = t % NB
        off = t * B + base
        if wb[s] is not None:
          wb[s].wait()
        if t == NB - 1:
          wq.wait()
        _extract_col(sb_v, 3 * t + 2, idx[s])
        gb[s] = pltpu.async_copy(t_ct.at[idx[s]], r64[s], gs[s])
        sm = t % 2
        if wm[sm] is not None:
          wm[sm].wait()
        _extract_col(sb_v, 3 * t, mkb[sm])
        wm[sm] = pltpu.async_copy(mkb[sm], mko.at[pl.ds(off, CH)], mss[sm])
        if t >= 1:
          sp = (t - 1) % NB
          gb[sp].wait()
          wb[sp] = pltpu.async_copy(
              r64[sp], bco.at[pl.ds((t - 1) * B + base, CH)], ws[sp])
      sp = (T - 1) % NB
      gb[sp].wait()
      wb[sp] = pltpu.async_copy(
          r64[sp], bco.at[pl.ds((T - 1) * B + base, CH)], ws[sp])
      for d in wb + wm:
        if d is not None:
          d.wait()
      return carry
    lax.fori_loop(0, nq, chunk, 0)

  return k(us, its, beh, emb_user_id, emb_user_city, emb_user_age, emb_cate)


def _sc_gather_b(its, beh, emb_item):
  B = its.shape[0]
  nq = (B // NW) // CH
  mesh = plsc.VectorSubcoreMesh(core_axis_name="c", subcore_axis_name="s")

  out_type = [
      jax.ShapeDtypeStruct((B, 64), jnp.float32),
      jax.ShapeDtypeStruct((T * B, 64), jnp.float32),
  ]
  scratch_types = (
      [pltpu.VMEM((CH, 3), jnp.int32),
       pltpu.VMEM((CH, 3 * T), jnp.int32)]
      + [pltpu.VMEM((CH,), jnp.int32) for _ in range(NB)]
      + [pltpu.VMEM((CH, 64), jnp.float32) for _ in range(NB)]
      + [pltpu.SemaphoreType.DMA for _ in range(2 * NB)]
  )

  @functools.partial(pl.kernel, out_type=out_type, mesh=mesh,
                     scratch_types=scratch_types, **_SC_PARAMS)
  def k(itsh, behh, t_it, qio, bio,
        s3_v, sb_v, i0, i1, i2, r640, r641, r642, g0, g1, g2, w0, w1, w2):
    idx = [i0, i1, i2]
    r64 = [r640, r641, r642]
    gs = [g0, g1, g2]
    ws = [w0, w1, w2]
    wid = lax.axis_index("s") * 2 + lax.axis_index("c")
    qbase = wid * (B // NW)

    def chunk(c, carry):
      base = qbase + c * CH
      pltpu.sync_copy(itsh.at[pl.ds(base, CH)], s3_v)
      _extract_col(s3_v, 0, idx[0])
      gq = pltpu.async_copy(t_it.at[idx[0]], r64[0], gs[0])
      pltpu.sync_copy(behh.at[pl.ds(base, CH)], sb_v)
      gq.wait()
      wq = pltpu.async_copy(r64[0], qio.at[pl.ds(base, CH)], ws[0])
      gb = [None] * NB
      wb = [None] * NB
      for t in range(T):
        s = t % NB
        if wb[s] is not None:
          wb[s].wait()
        if t == NB - 1:
          wq.wait()
        _extract_col(sb_v, 3 * t + 1, idx[s])
        gb[s] = pltpu.async_copy(t_it.at[idx[s]], r64[s], gs[s])
        if t >= 1:
          sp = (t - 1) % NB
          gb[sp].wait()
          wb[sp] = pltpu.async_copy(
              r64[sp], bio.at[pl.ds((t - 1) * B + base, CH)], ws[sp])
      sp = (T - 1) % NB
      gb[sp].wait()
      wb[sp] = pltpu.async_copy(
          r64[sp], bio.at[pl.ds((T - 1) * B + base, CH)], ws[sp])
      for d in wb:
        if d is not None:
          d.wait()
      return carry
    lax.fori_loop(0, nq, chunk, 0)

  return k(its, beh, emb_item)


def _prelu(x, a):
  return jnp.where(x >= 0.0, x, a * x)


def _dot(x, w):
  return jnp.dot(x, w, preferred_element_type=jnp.float32)


def _tc_body(Bb,
             ud_r, isf_r, ue0_r, ue1_r, ue2_r, qi_r, qc_r,
             bi_r, bc_r, mk_r,
             WAi_r, WAc_r, WBi_r, WBc_r, WDi_r, WDc_r,
             ab0_r, aa0_r, aW1_r, ab1_r, aa1_r, aWf_r, abf_r,
             Fud_r, Fisf_r, Fue0_r, Fue1_r, Fue2_r, Fqi_r, Fqc_r,
             Fai_r, Fac_r, fb0_r, fa0_r, fW1_r, fb1_r, fa1_r,
             oW_r, ob_r, out_r):
  qi = qi_r[...]
  qc = qc_r[...]
  bi3 = bi_r[...]
  bc3 = bc_r[...]

  bir = bi3.reshape(T * Bb, 64)
  bcr = bc3.reshape(T * Bb, 64)
  pir = (bi3 * qi[None, :, :]).reshape(T * Bb, 64)
  pcr = (bc3 * qc[None, :, :]).reshape(T * Bb, 64)

  hq = _dot(qi, WAi_r[...]) + _dot(qc, WAc_r[...])
  h0 = (jnp.broadcast_to(hq[None], (T, Bb, 80)).reshape(T * Bb, 80)
        + _dot(bir, WBi_r[...]) + _dot(bcr, WBc_r[...])
        + _dot(pir, WDi_r[...]) + _dot(pcr, WDc_r[...]) + ab0_r[...])
  h0 = _prelu(h0, aa0_r[...])
  h1 = _prelu(_dot(h0, aW1_r[...]) + ab1_r[...], aa1_r[...])
  s = _dot(h1, aWf_r[...]) + abf_r[...]
  s3 = s.reshape(T, Bb, 1)
  s3 = jnp.where(mk_r[...] == 0, jnp.float32(-4294967295.0), s3)
  m = jnp.max(s3, axis=0, keepdims=True)
  e = jnp.exp(s3 - m)
  w3 = e / jnp.sum(e, axis=0, keepdims=True)
  atti = jnp.sum(w3 * bi3, axis=0)
  attc = jnp.sum(w3 * bc3, axis=0)

  h2 = (_dot(ud_r[...], Fud_r[...]) + _dot(isf_r[...], Fisf_r[...])
        + _dot(ue0_r[...], Fue0_r[...]) + _dot(ue1_r[...], Fue1_r[...])
        + _dot(ue2_r[...], Fue2_r[...])
        + _dot(qi, Fqi_r[...]) + _dot(qc, Fqc_r[...])
        + _dot(atti, Fai_r[...]) + _dot(attc, Fac_r[...]) + fb0_r[...])
  h2 = _prelu(h2, fa0_r[...])
  h3 = _prelu(_dot(h2, fW1_r[...]) + fb1_r[...], fa1_r[...])
  out_r[...] = jax.nn.sigmoid(_dot(h3, oW_r[...]) + ob_r[...])


def _tc_dense(ud, isf, ue0, ue1, ue2, qi, qc, bi3, bc3, mk3, weights,
              interpret=False):
  B = ud.shape[0]
  Bb = 512 if B % 512 == 0 else B
  grid = (B // Bb,)

  def rows(n):
    return pl.BlockSpec((Bb, n), lambda i: (i, 0))

  def full(a):
    return pl.BlockSpec(a.shape, lambda i: (0,) * a.ndim)

  in_specs = [
      rows(5), rows(3), rows(32), rows(32), rows(32), rows(64), rows(64),
      pl.BlockSpec((T, Bb, 64), lambda i: (0, i, 0)),
      pl.BlockSpec((T, Bb, 64), lambda i: (0, i, 0)),
      pl.BlockSpec((T, Bb, 1), lambda i: (0, i, 0)),
  ] + [full(w) for w in weights]

  return pl.pallas_call(
      functools.partial(_tc_body, Bb),
      grid=grid,
      in_specs=in_specs,
      out_specs=pl.BlockSpec((Bb, 1), lambda i: (i, 0)),
      out_shape=jax.ShapeDtypeStruct((B, 1), jnp.float32),
      interpret=interpret,
  )(ud, isf, ue0, ue1, ue2, qi, qc, bi3, bc3, mk3, *weights)


def _prep_weights(att_W0, att_b0, att_a0, att_W1, att_b1, att_a1,
                  att_Wf, att_bf, bn_gamma, bn_beta,
                  ffn_W0, ffn_b0, ffn_a0, ffn_W1, ffn_b1, ffn_a1,
                  out_W, out_b):
  A, Bm, C, D = (att_W0[0:128], att_W0[128:256],
                 att_W0[256:384], att_W0[384:512])
  AC = A + C
  BC = Bm - C
  g = bn_gamma / math.sqrt(1.0 + 1e-3)
  F = ffn_W0 * g[:, None]
  fb0 = ffn_b0 + bn_beta @ ffn_W0
  r = lambda v: v.reshape(1, -1)
  return [
      AC[0:64], AC[64:128], BC[0:64], BC[64:128], D[0:64], D[64:128],
      r(att_b0), r(att_a0), att_W1, r(att_b1), r(att_a1), att_Wf, r(att_bf),
      F[0:5], F[101:104], F[5:37], F[37:69], F[69:101],
      F[104:168], F[168:232], F[232:296], F[296:360],
      r(fb0), r(ffn_a0), ffn_W1, r(ffn_b1), r(ffn_a1), out_W, r(out_b),
  ]


def kernel(user_dense_input, user_sparse_input, item_dense_input,
           item_sparse_input, behavior_input, emb_user_id, emb_user_city,
           emb_user_age, emb_item, emb_cate, att_W0, att_b0, att_a0,
           att_W1, att_b1, att_a1, att_Wf, att_bf, bn_gamma, bn_beta,
           ffn_W0, ffn_b0, ffn_a0, ffn_W1, ffn_b1, ffn_a1, out_W, out_b):
  B = user_dense_input.shape[0]
  us = user_sparse_input.astype(jnp.int32)
  its = item_sparse_input.astype(jnp.int32)
  beh = behavior_input.astype(jnp.int32)

  ue0, ue1, ue2, qc, bc, mk = _sc_gather_a(
      us, its, beh, emb_user_id, emb_user_city, emb_user_age, emb_cate)
  qi, bi = _sc_gather_b(its, beh, emb_item)
  mk3 = mk.reshape(T, B, 1)

  weights = _prep_weights(att_W0, att_b0, att_a0, att_W1, att_b1, att_a1,
                          att_Wf, att_bf, bn_gamma, bn_beta,
                          ffn_W0, ffn_b0, ffn_a0, ffn_W1, ffn_b1, ffn_a1,
                          out_W, out_b)

  return _tc_dense(user_dense_input, its.astype(jnp.float32),
                   ue0, ue1, ue2, qi, qc,
                   bi.reshape(T, B, 64), bc.reshape(T, B, 64), mk3, weights)


# speedup vs baseline: 1.5226x; 1.4067x over previous
import functools
import math

import jax
import jax.numpy as jnp
from jax import lax
from jax.experimental import pallas as pl
from jax.experimental.pallas import tpu as pltpu
from jax.experimental.pallas import tpu_sc as plsc

T = 20
NW = 32
CH = 128


_SC_PARAMS = dict(
    compiler_params=pltpu.CompilerParams(
        use_tc_tiling_on_sc=False, needs_layout_passes=False))
NB = 3


def _extract_col(src_v, col, dst_v):
  for g in range(CH // 16):
    rows = lax.iota(jnp.int32, 16) + g * 16
    cols = jnp.broadcast_to(jnp.int32(0) + col, (16,))
    dst_v[pl.ds(g * 16, 16)] = plsc.load_gather(src_v, [rows, cols])


def _sc_gather_a(us, its, beh,
                 emb_user_id, emb_user_city, emb_user_age, emb_cate):
  B = us.shape[0]
  nq = (B // NW) // CH
  mesh = plsc.VectorSubcoreMesh(core_axis_name="c", subcore_axis_name="s")

  out_type = [
      jax.ShapeDtypeStruct((B, 32), jnp.float32),
      jax.ShapeDtypeStruct((B, 32), jnp.float32),
      jax.ShapeDtypeStruct((B, 32), jnp.float32),
      jax.ShapeDtypeStruct((B, 64), jnp.float32),
      jax.ShapeDtypeStruct((T * B,), jnp.int32),
  ]
  scratch_types = (
      [pltpu.VMEM((CH, 3), jnp.int32),
       pltpu.VMEM((CH, 3 * T), jnp.int32)]
      + [pltpu.VMEM((CH,), jnp.int32) for _ in range(NB)]
      + [pltpu.VMEM((CH, 32), jnp.float32) for _ in range(NB)]
      + [pltpu.VMEM((CH, 64), jnp.float32)]
      + [pltpu.VMEM((CH,), jnp.int32) for _ in range(2)]
      + [pltpu.SemaphoreType.DMA for _ in range(2 * NB + 3)]
  )

  @functools.partial(pl.kernel, out_type=out_type, mesh=mesh,
                     scratch_types=scratch_types, **_SC_PARAMS)
  def k(ush, itsh, behh, t_u0, t_u1, t_u2, t_ct,
        ue0o, ue1o, ue2o, qco, mko,
        s3_v, sb_v, i0, i1, i2, r320, r321, r322, r64q,
        m0, m1, g0, g1, g2, w0, w1, w2, ms0, ms1, qw):
    idx = [i0, i1, i2]
    r32 = [r320, r321, r322]
    mkb = [m0, m1]
    gs = [g0, g1, g2]
    ws = [w0, w1, w2]
    mss = [ms0, ms1]
    wid = lax.axis_index("s") * 2 + lax.axis_index("c")
    qbase = wid * (B // NW)

    def chunk(c, carry):
      base = qbase + c * CH
      pltpu.sync_copy(ush.at[pl.ds(base, CH)], s3_v)
      tabs = [(t_u0, ue0o), (t_u1, ue1o), (t_u2, ue2o)]
      gd = []
      for f, (tab, _) in enumerate(tabs):
        _extract_col(s3_v, f, idx[f])
        gd.append(pltpu.async_copy(tab.at[idx[f]], r32[f], gs[f]))
      pltpu.sync_copy(itsh.at[pl.ds(base, CH)], s3_v)
      _extract_col(s3_v, 1, i2)
      gq = pltpu.async_copy(t_ct.at[i2], r64q, gs[2])
      pltpu.sync_copy(behh.at[pl.ds(base, CH)], sb_v)
      wd = []
      for f, (_, outh) in enumerate(tabs):
        gd[f].wait()
        wd.append(pltpu.async_copy(r32[f], outh.at[pl.ds(base, CH)], ws[f]))
      gq.wait()
      wq = pltpu.async_copy(r64q, qco.at[pl.ds(base, CH)], qw)
      wm = [None, None]
      for t in range(T):
        sm = t % 2
        if wm[sm] is not None:
          wm[sm].wait()
        _extract_col(sb_v, 3 * t, mkb[sm])
        wm[sm] = pltpu.async_copy(
            mkb[sm], mko.at[pl.ds(t * B + base, CH)], mss[sm])
      for d in wd + wm + [wq]:
        if d is not None:
          d.wait()
      return carry
    lax.fori_loop(0, nq, chunk, 0)

  return k(us, its, beh, emb_user_id, emb_user_city, emb_user_age, emb_cate)


def _sc_gather_b(its, beh, emb_item, emb_cate):
  B = its.shape[0]
  nq = (B // NW) // CH
  mesh = plsc.VectorSubcoreMesh(core_axis_name="c", subcore_axis_name="s")

  out_type = [
      jax.ShapeDtypeStruct((B, 64), jnp.float32),
      jax.ShapeDtypeStruct((T * B, 128), jnp.float32),
  ]
  scratch_types = (
      [pltpu.VMEM((CH, 3), jnp.int32),
       pltpu.VMEM((CH, 3 * T), jnp.int32)]
      + [pltpu.VMEM((CH,), jnp.int32) for _ in range(2 * NB)]
      + [pltpu.VMEM((CH, 64), jnp.float32) for _ in range(2 * NB)]
      + [pltpu.SemaphoreType.DMA for _ in range(4 * NB)]
  )

  @functools.partial(pl.kernel, out_type=out_type, mesh=mesh,
                     scratch_types=scratch_types, **_SC_PARAMS)
  def k(itsh, behh, t_it, t_ct, qio, bo,
        s3_v, sb_v, i0, i1, i2, i3, i4, i5,
        r0, r1, r2, r3, r4, r5,
        g0, g1, g2, g3, g4, g5, w0, w1, w2, w3, w4, w5):
    idxi = [i0, i1, i2]
    idxc = [i3, i4, i5]
    ri = [r0, r1, r2]
    rc = [r3, r4, r5]
    gsi = [g0, g1, g2]
    gsc = [g3, g4, g5]
    wsi = [w0, w1, w2]
    wsc = [w3, w4, w5]
    wid = lax.axis_index("s") * 2 + lax.axis_index("c")
    qbase = wid * (B // NW)

    def chunk(c, carry):
      base = qbase + c * CH
      pltpu.sync_copy(itsh.at[pl.ds(base, CH)], s3_v)
      _extract_col(s3_v, 0, i0)
      gq = pltpu.async_copy(t_it.at[i0], r0, g0)
      pltpu.sync_copy(behh.at[pl.ds(base, CH)], sb_v)
      gq.wait()
      wq = pltpu.async_copy(r0, qio.at[pl.ds(base, CH)], w0)
      gi = [None] * NB
      gc = [None] * NB
      wi = [None] * NB
      wc = [None] * NB
      for t in range(T):
        s = t % NB
        if wi[s] is not None:
          wi[s].wait()
          wc[s].wait()
        if t == 0:
          wq.wait()
        _extract_col(sb_v, 3 * t + 1, idxi[s])
        gi[s] = pltpu.async_copy(t_it.at[idxi[s]], ri[s], gsi[s])
        _extract_col(sb_v, 3 * t + 2, idxc[s])
        gc[s] = pltpu.async_copy(t_ct.at[idxc[s]], rc[s], gsc[s])
        if t >= 1:
          sp = (t - 1) % NB
          off = (t - 1) * B + base
          gi[sp].wait()
          wi[sp] = pltpu.async_copy(
              ri[sp], bo.at[pl.ds(off, CH), pl.ds(0, 64)], wsi[sp])
          gc[sp].wait()
          wc[sp] = pltpu.async_copy(
              rc[sp], bo.at[pl.ds(off, CH), pl.ds(64, 64)], wsc[sp])
      sp = (T - 1) % NB
      off = (T - 1) * B + base
      gi[sp].wait()
      wi[sp] = pltpu.async_copy(
          ri[sp], bo.at[pl.ds(off, CH), pl.ds(0, 64)], wsi[sp])
      gc[sp].wait()
      wc[sp] = pltpu.async_copy(
          rc[sp], bo.at[pl.ds(off, CH), pl.ds(64, 64)], wsc[sp])
      for d in wi + wc:
        if d is not None:
          d.wait()
      return carry
    lax.fori_loop(0, nq, chunk, 0)

  return k(its, beh, emb_item, emb_cate)


def _prelu(x, a):
  return jnp.where(x >= 0.0, x, a * x)


def _dot(x, w):
  return jnp.dot(x, w, preferred_element_type=jnp.float32)


def _tc_body(Bb,
             ud_r, isf_r, ue0_r, ue1_r, ue2_r, qi_r, qc_r,
             bk_r, mk_r,
             WAC_r, WBC_r, WD_r,
             ab0_r, aa0_r, aW1_r, ab1_r, aa1_r, aWf_r, abf_r,
             Fud_r, Fisf_r, Fue0_r, Fue1_r, Fue2_r, Fq_r, Fatt_r,
             fb0_r, fa0_r, fW1_r, fb1_r, fa1_r,
             oW_r, ob_r, out_r):
  q = jnp.concatenate([qi_r[...], qc_r[...]], axis=-1)
  k3 = bk_r[...]

  kr = k3.reshape(T * Bb, 128)
  pr = (k3 * q[None, :, :]).reshape(T * Bb, 128)

  hq = _dot(q, WAC_r[...])
  h0 = (jnp.broadcast_to(hq[None], (T, Bb, 80)).reshape(T * Bb, 80)
        + _dot(kr, WBC_r[...]) + _dot(pr, WD_r[...]) + ab0_r[...])
  h0 = _prelu(h0, aa0_r[...])
  h1 = _prelu(_dot(h0, aW1_r[...]) + ab1_r[...], aa1_r[...])
  s = _dot(h1, aWf_r[...]) + abf_r[...]
  s3 = s.reshape(T, Bb, 1)
  mk3 = mk_r[...][:, :, None]
  s3 = jnp.where(mk3 == 0, jnp.float32(-4294967295.0), s3)
  m = jnp.max(s3, axis=0, keepdims=True)
  e = jnp.exp(s3 - m)
  w3 = e / jnp.sum(e, axis=0, keepdims=True)
  att = jnp.sum(w3 * k3, axis=0)

  h2 = (_dot(ud_r[...], Fud_r[...]) + _dot(isf_r[...], Fisf_r[...])
        + _dot(ue0_r[...], Fue0_r[...]) + _dot(ue1_r[...], Fue1_r[...])
        + _dot(ue2_r[...], Fue2_r[...])
        + _dot(q, Fq_r[...]) + _dot(att, Fatt_r[...]) + fb0_r[...])
  h2 = _prelu(h2, fa0_r[...])
  h3 = _prelu(_dot(h2, fW1_r[...]) + fb1_r[...], fa1_r[...])
  out_r[...] = jax.nn.sigmoid(_dot(h3, oW_r[...]) + ob_r[...])


def _tc_dense(ud, isf, ue0, ue1, ue2, qi, qc, bk3, mk2, weights,
              interpret=False):
  B = ud.shape[0]
  Bb = 512 if B % 512 == 0 else B
  grid = (B // Bb,)

  def rows(n):
    return pl.BlockSpec((Bb, n), lambda i: (i, 0))

  def full(a):
    return pl.BlockSpec(a.shape, lambda i: (0,) * a.ndim)

  in_specs = [
      rows(5), rows(3), rows(32), rows(32), rows(32), rows(64), rows(64),
      pl.BlockSpec((T, Bb, 128), lambda i: (0, i, 0)),
      pl.BlockSpec((T, Bb), lambda i: (0, i)),
  ] + [full(w) for w in weights]

  return pl.pallas_call(
      functools.partial(_tc_body, Bb),
      grid=grid,
      in_specs=in_specs,
      out_specs=pl.BlockSpec((Bb, 1), lambda i: (i, 0)),
      out_shape=jax.ShapeDtypeStruct((B, 1), jnp.float32),
      interpret=interpret,
  )(ud, isf, ue0, ue1, ue2, qi, qc, bk3, mk2, *weights)


def _prep_weights(att_W0, att_b0, att_a0, att_W1, att_b1, att_a1,
                  att_Wf, att_bf, bn_gamma, bn_beta,
                  ffn_W0, ffn_b0, ffn_a0, ffn_W1, ffn_b1, ffn_a1,
                  out_W, out_b):
  A, Bm, C, D = (att_W0[0:128], att_W0[128:256],
                 att_W0[256:384], att_W0[384:512])
  g = bn_gamma / math.sqrt(1.0 + 1e-3)
  F = ffn_W0 * g[:, None]
  fb0 = ffn_b0 + bn_beta @ ffn_W0
  r = lambda v: v.reshape(1, -1)
  return [
      A + C, Bm - C, D,
      r(att_b0), r(att_a0), att_W1, r(att_b1), r(att_a1), att_Wf, r(att_bf),
      F[0:5], F[101:104], F[5:37], F[37:69], F[69:101],
      F[104:232], F[232:360],
      r(fb0), r(ffn_a0), ffn_W1, r(ffn_b1), r(ffn_a1), out_W, r(out_b),
  ]


def kernel(user_dense_input, user_sparse_input, item_dense_input,
           item_sparse_input, behavior_input, emb_user_id, emb_user_city,
           emb_user_age, emb_item, emb_cate, att_W0, att_b0, att_a0,
           att_W1, att_b1, att_a1, att_Wf, att_bf, bn_gamma, bn_beta,
           ffn_W0, ffn_b0, ffn_a0, ffn_W1, ffn_b1, ffn_a1, out_W, out_b):
  B = user_dense_input.shape[0]
  us = user_sparse_input.astype(jnp.int32)
  its = item_sparse_input.astype(jnp.int32)
  beh = behavior_input.astype(jnp.int32)

  ue0, ue1, ue2, qc, mk = _sc_gather_a(
      us, its, beh, emb_user_id, emb_user_city, emb_user_age, emb_cate)
  qi, bk = _sc_gather_b(its, beh, emb_item, emb_cate)
  mk2 = mk.reshape(T, B)

  weights = _prep_weights(att_W0, att_b0, att_a0, att_W1, att_b1, att_a1,
                          att_Wf, att_bf, bn_gamma, bn_beta,
                          ffn_W0, ffn_b0, ffn_a0, ffn_W1, ffn_b1, ffn_a1,
                          out_W, out_b)

  return _tc_dense(user_dense_input, its.astype(jnp.float32),
                   ue0, ue1, ue2, qi, qc,
                   bk.reshape(T, B, 128), mk2, weights)
